# baseline (speedup 1.0000x reference)
import functools

import jax
import jax.numpy as jnp
from jax import lax
from jax.experimental import pallas as pl
from jax.experimental.pallas import tpu as pltpu
from jax.experimental.pallas import tpu_sc as plsc

N = 10000
NP = 10000
E = 320000
D = 128

NC = 2
NS = 16
NW = NC * NS
EPW = E // NW
CHUNK = 80
NCHUNK = EPW // CHUNK
RPS = NP // NS
CW = 16
CHUNKC = 125
NCHUNKC = EPW // CHUNKC

_MESH = plsc.VectorSubcoreMesh(
    core_axis_name="c", subcore_axis_name="s", num_cores=NC)


def _segsum_body(feat, srcs, dsts, zfeat, out_sum, srcv, dstv, rows, acc, sem):
    c = lax.axis_index("c")
    s = lax.axis_index("s")
    wid = c * NS + s
    pltpu.sync_copy(srcs.at[wid], srcv)
    pltpu.sync_copy(dsts.at[wid], dstv)
    pltpu.sync_copy(zfeat, acc.at[pl.ds(s * RPS, RPS)])
    plsc.subcore_barrier()

    def step(j, carry):
        @pl.when(j < NCHUNK)
        def _():
            pltpu.async_copy(feat.at[srcv.at[j]], rows.at[j % 3], sem)

        @pl.when(j >= 2)
        def _():
            jm = j - 2
            pltpu.make_async_copy(
                feat.at[srcv.at[jm]], rows.at[jm % 3], sem).wait()
            pltpu.sync_copy(rows.at[jm % 3], acc.at[dstv.at[jm]], add=True)
        return carry

    lax.fori_loop(0, NCHUNK + 2, step, 0)
    plsc.subcore_barrier()
    pltpu.sync_copy(acc.at[pl.ds(s * RPS, RPS)],
                    out_sum.at[c, pl.ds(s * RPS, RPS)])


_segsum = pl.kernel(
    _segsum_body,
    out_type=jax.ShapeDtypeStruct((NC, NP, D), jnp.float32),
    mesh=_MESH,
    scratch_types=[
        pltpu.VMEM((NCHUNK, CHUNK), jnp.int32),
        pltpu.VMEM((NCHUNK, CHUNK), jnp.int32),
        pltpu.VMEM((3, CHUNK, D), jnp.float32),
        pltpu.VMEM_SHARED((NP, D), jnp.float32),
        pltpu.SemaphoreType.DMA,
    ],
    compiler_params=pltpu.CompilerParams(use_tc_tiling_on_sc=False),
)


def _segcnt_body(dsts, zcnt, ones, out_cnt, dstv, onesv, cacc):
    c = lax.axis_index("c")
    s = lax.axis_index("s")
    wid = c * NS + s
    pltpu.sync_copy(dsts.at[wid], dstv)
    pltpu.sync_copy(zcnt, cacc.at[pl.ds(s * RPS, RPS)])
    pltpu.sync_copy(ones, onesv)
    plsc.subcore_barrier()

    def step(j, carry):
        pltpu.sync_copy(onesv, cacc.at[dstv.at[j]], add=True)
        return carry

    lax.fori_loop(0, NCHUNKC, step, 0)
    plsc.subcore_barrier()
    pltpu.sync_copy(cacc.at[pl.ds(s * RPS, RPS)],
                    out_cnt.at[c, pl.ds(s * RPS, RPS)])


_segcnt = pl.kernel(
    _segcnt_body,
    out_type=jax.ShapeDtypeStruct((NC, NP, CW), jnp.float32),
    mesh=_MESH,
    scratch_types=[
        pltpu.VMEM((NCHUNKC, CHUNKC), jnp.int32),
        pltpu.VMEM((CHUNKC, CW), jnp.float32),
        pltpu.VMEM_SHARED((NP, CW), jnp.float32),
    ],
    compiler_params=pltpu.CompilerParams(use_tc_tiling_on_sc=False),
)

R = 10000


def _dense1_body(x_ref, s_ref, c_ref, wl_ref, b_ref, wr_ref,
                 g_ref, bt_ref, mu_ref, var_ref, h_ref):
    ssum = s_ref[0] + s_ref[1]
    cnt = c_ref[0, :, 0:1] + c_ref[1, :, 0:1]
    agg = ssum / jnp.maximum(cnt, 1.0)
    h = lax.dot_general(agg, wl_ref[...], (((1,), (1,)), ((), ())),
                        preferred_element_type=jnp.float32)
    h = h + b_ref[...]
    h = h + lax.dot_general(x_ref[...], wr_ref[...], (((1,), (1,)), ((), ())),
                            preferred_element_type=jnp.float32)
    scale = g_ref[...] * lax.rsqrt(var_ref[...] + 1e-5)
    h = (h - mu_ref[...]) * scale + bt_ref[...]
    h_ref[...] = jnp.maximum(h, 0.0)


def _dense2_body(h_ref, s_ref, c_ref, wl_ref, b_ref, wr_ref, o_ref):
    ssum = s_ref[0] + s_ref[1]
    cnt = c_ref[0, :, 0:1] + c_ref[1, :, 0:1]
    agg = ssum / jnp.maximum(cnt, 1.0)
    o = lax.dot_general(agg, wl_ref[...], (((1,), (1,)), ((), ())),
                        preferred_element_type=jnp.float32)
    o = o + b_ref[...]
    o = o + lax.dot_general(h_ref[...], wr_ref[...], (((1,), (1,)), ((), ())),
                            preferred_element_type=jnp.float32)
    o_ref[...] = o


_row_spec = pl.BlockSpec((R, D), lambda i: (i, 0))
_part_spec = pl.BlockSpec((NC, R, D), lambda i: (0, i, 0))
_cnt_spec = pl.BlockSpec((NC, R, CW), lambda i: (0, i, 0))
_w_spec = pl.BlockSpec((D, D), lambda i: (0, 0))
_v_spec = pl.BlockSpec((1, D), lambda i: (0, 0))

_dense1 = pl.pallas_call(
    _dense1_body,
    grid=(N // R,),
    in_specs=[_row_spec, _part_spec, _cnt_spec, _w_spec, _v_spec, _w_spec,
              _v_spec, _v_spec, _v_spec, _v_spec],
    out_specs=_row_spec,
    out_shape=jax.ShapeDtypeStruct((N, D), jnp.float32),
)

_dense2 = pl.pallas_call(
    _dense2_body,
    grid=(N // R,),
    in_specs=[_row_spec, _part_spec, _cnt_spec, _w_spec, _v_spec, _w_spec],
    out_specs=_row_spec,
    out_shape=jax.ShapeDtypeStruct((N, D), jnp.float32),
)


def kernel(x, edge_index, W1l, b1, W1r, gamma, beta, run_mean, run_var,
           W2l, b2, W2r):
    src = edge_index[0].reshape(NW, NCHUNK, CHUNK)
    dst = edge_index[1].reshape(NW, NCHUNK, CHUNK)
    zfeat = jnp.zeros((RPS, D), jnp.float32)
    dstc = edge_index[1].reshape(NW, NCHUNKC, CHUNKC)
    zcnt = jnp.zeros((RPS, CW), jnp.float32)
    ones = jnp.ones((CHUNKC, CW), jnp.float32)

    c1 = _segcnt(dstc, zcnt, ones)
    s1 = _segsum(x, src, dst, zfeat)
    h = _dense1(x, s1, c1, W1l, b1.reshape(1, D), W1r,
                gamma.reshape(1, D), beta.reshape(1, D),
                run_mean.reshape(1, D), run_var.reshape(1, D))
    s2 = _segsum(h, src, dst, zfeat)
    out = _dense2(h, s2, c1, W2l, b2.reshape(1, D), W2r)
    return out


# speedup vs baseline: 1.0126x; 1.0126x over previous
import functools

import jax
import jax.numpy as jnp
from jax import lax
from jax.experimental import pallas as pl
from jax.experimental.pallas import tpu as pltpu
from jax.experimental.pallas import tpu_sc as plsc

N = 10000
NP = 10000
E = 320000
D = 128

NC = 2
NS = 16
NW = NC * NS
EPW = E // NW
CHUNK = 80
NCHUNK = EPW // CHUNK
RPS = NP // NS
CW = 16
CHUNKC = 125
NCHUNKC = EPW // CHUNKC

_MESH = plsc.VectorSubcoreMesh(
    core_axis_name="c", subcore_axis_name="s", num_cores=NC)


def _segsum_body(feat, srcs, dsts, zfeat, out_sum, srcv, dstv, rows, acc, sem):
    c = lax.axis_index("c")
    s = lax.axis_index("s")
    wid = c * NS + s
    cp_s = pltpu.async_copy(srcs.at[wid], srcv, sem)
    cp_d = pltpu.async_copy(dsts.at[wid], dstv, sem)
    cp_z = pltpu.async_copy(zfeat, acc.at[pl.ds(s * RPS, RPS)], sem)
    cp_s.wait()
    cp_d.wait()
    cp_z.wait()
    plsc.subcore_barrier()

    def step(j, carry):
        @pl.when(j < NCHUNK)
        def _():
            pltpu.async_copy(feat.at[srcv.at[j]], rows.at[j % 3], sem)

        @pl.when(j >= 2)
        def _():
            jm = j - 2
            pltpu.make_async_copy(
                feat.at[srcv.at[jm]], rows.at[jm % 3], sem).wait()
            pltpu.sync_copy(rows.at[jm % 3], acc.at[dstv.at[jm]], add=True)
        return carry

    lax.fori_loop(0, NCHUNK + 2, step, 0)
    plsc.subcore_barrier()
    pltpu.sync_copy(acc.at[pl.ds(s * RPS, RPS)],
                    out_sum.at[c, pl.ds(s * RPS, RPS)])


_segsum = pl.kernel(
    _segsum_body,
    out_type=jax.ShapeDtypeStruct((NC, NP, D), jnp.float32),
    mesh=_MESH,
    scratch_types=[
        pltpu.VMEM((NCHUNK, CHUNK), jnp.int32),
        pltpu.VMEM((NCHUNK, CHUNK), jnp.int32),
        pltpu.VMEM((3, CHUNK, D), jnp.float32),
        pltpu.VMEM_SHARED((NP, D), jnp.float32),
        pltpu.SemaphoreType.DMA,
    ],
    compiler_params=pltpu.CompilerParams(use_tc_tiling_on_sc=False),
)


def _segcnt_body(dsts, zcnt, ones, out_cnt, dstv, onesv, cacc):
    c = lax.axis_index("c")
    s = lax.axis_index("s")
    wid = c * NS + s
    pltpu.sync_copy(dsts.at[wid], dstv)
    pltpu.sync_copy(zcnt, cacc.at[pl.ds(s * RPS, RPS)])
    pltpu.sync_copy(ones, onesv)
    plsc.subcore_barrier()

    def step(j, carry):
        pltpu.sync_copy(onesv, cacc.at[dstv.at[j]], add=True)
        return carry

    lax.fori_loop(0, NCHUNKC, step, 0)
    plsc.subcore_barrier()
    pltpu.sync_copy(cacc.at[pl.ds(s * RPS, RPS)],
                    out_cnt.at[c, pl.ds(s * RPS, RPS)])


_segcnt = pl.kernel(
    _segcnt_body,
    out_type=jax.ShapeDtypeStruct((NC, NP, CW), jnp.float32),
    mesh=_MESH,
    scratch_types=[
        pltpu.VMEM((NCHUNKC, CHUNKC), jnp.int32),
        pltpu.VMEM((CHUNKC, CW), jnp.float32),
        pltpu.VMEM_SHARED((NP, CW), jnp.float32),
    ],
    compiler_params=pltpu.CompilerParams(use_tc_tiling_on_sc=False),
)

R = 2000


def _dense1_body(x_ref, s_ref, c_ref, wl_ref, b_ref, wr_ref,
                 g_ref, bt_ref, mu_ref, var_ref, h_ref):
    ssum = s_ref[0] + s_ref[1]
    cnt = c_ref[0, :, 0:1] + c_ref[1, :, 0:1]
    agg = ssum / jnp.maximum(cnt, 1.0)
    h = lax.dot_general(agg, wl_ref[...], (((1,), (1,)), ((), ())),
                        preferred_element_type=jnp.float32)
    h = h + b_ref[...]
    h = h + lax.dot_general(x_ref[...], wr_ref[...], (((1,), (1,)), ((), ())),
                            preferred_element_type=jnp.float32)
    scale = g_ref[...] * lax.rsqrt(var_ref[...] + 1e-5)
    h = (h - mu_ref[...]) * scale + bt_ref[...]
    h_ref[...] = jnp.maximum(h, 0.0)


def _dense2_body(h_ref, s_ref, c_ref, wl_ref, b_ref, wr_ref, o_ref):
    ssum = s_ref[0] + s_ref[1]
    cnt = c_ref[0, :, 0:1] + c_ref[1, :, 0:1]
    agg = ssum / jnp.maximum(cnt, 1.0)
    o = lax.dot_general(agg, wl_ref[...], (((1,), (1,)), ((), ())),
                        preferred_element_type=jnp.float32)
    o = o + b_ref[...]
    o = o + lax.dot_general(h_ref[...], wr_ref[...], (((1,), (1,)), ((), ())),
                            preferred_element_type=jnp.float32)
    o_ref[...] = o


_row_spec = pl.BlockSpec((R, D), lambda i: (i, 0))
_part_spec = pl.BlockSpec((NC, R, D), lambda i: (0, i, 0))
_cnt_spec = pl.BlockSpec((NC, R, CW), lambda i: (0, i, 0))
_w_spec = pl.BlockSpec((D, D), lambda i: (0, 0))
_v_spec = pl.BlockSpec((1, D), lambda i: (0, 0))

_dense1 = pl.pallas_call(
    _dense1_body,
    grid=(N // R,),
    in_specs=[_row_spec, _part_spec, _cnt_spec, _w_spec, _v_spec, _w_spec,
              _v_spec, _v_spec, _v_spec, _v_spec],
    out_specs=_row_spec,
    out_shape=jax.ShapeDtypeStruct((N, D), jnp.float32),
)

_dense2 = pl.pallas_call(
    _dense2_body,
    grid=(N // R,),
    in_specs=[_row_spec, _part_spec, _cnt_spec, _w_spec, _v_spec, _w_spec],
    out_specs=_row_spec,
    out_shape=jax.ShapeDtypeStruct((N, D), jnp.float32),
)


def kernel(x, edge_index, W1l, b1, W1r, gamma, beta, run_mean, run_var,
           W2l, b2, W2r):
    src = edge_index[0].reshape(NW, NCHUNK, CHUNK)
    dst = edge_index[1].reshape(NW, NCHUNK, CHUNK)
    zfeat = jnp.zeros((RPS, D), jnp.float32)
    dstc = edge_index[1].reshape(NW, NCHUNKC, CHUNKC)
    zcnt = jnp.zeros((RPS, CW), jnp.float32)
    ones = jnp.ones((CHUNKC, CW), jnp.float32)

    c1 = _segcnt(dstc, zcnt, ones)
    s1 = _segsum(x, src, dst, zfeat)
    h = _dense1(x, s1, c1, W1l, b1.reshape(1, D), W1r,
                gamma.reshape(1, D), beta.reshape(1, D),
                run_mean.reshape(1, D), run_var.reshape(1, D))
    s2 = _segsum(h, src, dst, zfeat)
    out = _dense2(h, s2, c1, W2l, b2.reshape(1, D), W2r)
    return out


# speedup vs baseline: 1.0181x; 1.0054x over previous
import functools

import jax
import jax.numpy as jnp
from jax import lax
from jax.experimental import pallas as pl
from jax.experimental.pallas import tpu as pltpu
from jax.experimental.pallas import tpu_sc as plsc

N = 10000
NP = 10000
E = 320000
D = 128

NC = 2
NS = 16
NW = NC * NS
EPW = E // NW
CHUNK = 80
NCHUNK = EPW // CHUNK
RPS = NP // NS
CW = 16
CHUNKC = 125
NCHUNKC = EPW // CHUNKC

_MESH = plsc.VectorSubcoreMesh(
    core_axis_name="c", subcore_axis_name="s", num_cores=NC)


def _segsum_body(feat, srcs, dsts, zfeat, out_sum, srcv, dstv, rows, acc, sem):
    c = lax.axis_index("c")
    s = lax.axis_index("s")
    wid = c * NS + s
    cp_s = pltpu.async_copy(srcs.at[wid], srcv, sem)
    cp_d = pltpu.async_copy(dsts.at[wid], dstv, sem)
    cp_z = pltpu.async_copy(zfeat, acc.at[pl.ds(s * RPS, RPS)], sem)
    cp_s.wait()
    cp_d.wait()
    cp_z.wait()
    plsc.subcore_barrier()

    def step(j, carry):
        @pl.when(j < NCHUNK)
        def _():
            pltpu.async_copy(feat.at[srcv.at[j]], rows.at[j % 3], sem)

        @pl.when(j >= 2)
        def _():
            jm = j - 2
            pltpu.make_async_copy(
                feat.at[srcv.at[jm]], rows.at[jm % 3], sem).wait()
            pltpu.sync_copy(rows.at[jm % 3], acc.at[dstv.at[jm]], add=True)
        return carry

    lax.fori_loop(0, NCHUNK + 2, step, 0)
    plsc.subcore_barrier()
    pltpu.sync_copy(acc.at[pl.ds(s * RPS, RPS)],
                    out_sum.at[c, pl.ds(s * RPS, RPS)])


_segsum = pl.kernel(
    _segsum_body,
    out_type=jax.ShapeDtypeStruct((NC, NP, D), jnp.float32),
    mesh=_MESH,
    scratch_types=[
        pltpu.VMEM((NCHUNK, CHUNK), jnp.int32),
        pltpu.VMEM((NCHUNK, CHUNK), jnp.int32),
        pltpu.VMEM((3, CHUNK, D), jnp.float32),
        pltpu.VMEM_SHARED((NP, D), jnp.float32),
        pltpu.SemaphoreType.DMA,
    ],
    compiler_params=pltpu.CompilerParams(use_tc_tiling_on_sc=False),
)


def _segcnt_body(dsts, zcnt, ones, out_cnt, dstv, onesv, cacc, csem):
    c = lax.axis_index("c")
    s = lax.axis_index("s")
    wid = c * NS + s
    cp_d = pltpu.async_copy(dsts.at[wid], dstv, csem)
    cp_z = pltpu.async_copy(zcnt, cacc.at[pl.ds(s * RPS, RPS)], csem)
    cp_o = pltpu.async_copy(ones, onesv, csem)
    cp_d.wait()
    cp_z.wait()
    cp_o.wait()
    plsc.subcore_barrier()

    def step(j, carry):
        pltpu.sync_copy(onesv, cacc.at[dstv.at[j]], add=True)
        return carry

    lax.fori_loop(0, NCHUNKC, step, 0)
    plsc.subcore_barrier()
    pltpu.sync_copy(cacc.at[pl.ds(s * RPS, RPS)],
                    out_cnt.at[c, pl.ds(s * RPS, RPS)])


_segcnt = pl.kernel(
    _segcnt_body,
    out_type=jax.ShapeDtypeStruct((NC, NP, CW), jnp.float32),
    mesh=_MESH,
    scratch_types=[
        pltpu.VMEM((NCHUNKC, CHUNKC), jnp.int32),
        pltpu.VMEM((CHUNKC, CW), jnp.float32),
        pltpu.VMEM_SHARED((NP, CW), jnp.float32),
        pltpu.SemaphoreType.DMA,
    ],
    compiler_params=pltpu.CompilerParams(use_tc_tiling_on_sc=False),
)

R = 2000


def _dense1_body(x_ref, s_ref, c_ref, wl_ref, b_ref, wr_ref,
                 g_ref, bt_ref, mu_ref, var_ref, h_ref):
    ssum = s_ref[0] + s_ref[1]
    cnt = c_ref[0, :, 0:1] + c_ref[1, :, 0:1]
    agg = ssum / jnp.maximum(cnt, 1.0)
    h = lax.dot_general(agg, wl_ref[...], (((1,), (1,)), ((), ())),
                        preferred_element_type=jnp.float32)
    h = h + b_ref[...]
    h = h + lax.dot_general(x_ref[...], wr_ref[...], (((1,), (1,)), ((), ())),
                            preferred_element_type=jnp.float32)
    scale = g_ref[...] * lax.rsqrt(var_ref[...] + 1e-5)
    h = (h - mu_ref[...]) * scale + bt_ref[...]
    h_ref[...] = jnp.maximum(h, 0.0)


def _dense2_body(h_ref, s_ref, c_ref, wl_ref, b_ref, wr_ref, o_ref):
    ssum = s_ref[0] + s_ref[1]
    cnt = c_ref[0, :, 0:1] + c_ref[1, :, 0:1]
    agg = ssum / jnp.maximum(cnt, 1.0)
    o = lax.dot_general(agg, wl_ref[...], (((1,), (1,)), ((), ())),
                        preferred_element_type=jnp.float32)
    o = o + b_ref[...]
    o = o + lax.dot_general(h_ref[...], wr_ref[...], (((1,), (1,)), ((), ())),
                            preferred_element_type=jnp.float32)
    o_ref[...] = o


_row_spec = pl.BlockSpec((R, D), lambda i: (i, 0))
_part_spec = pl.BlockSpec((NC, R, D), lambda i: (0, i, 0))
_cnt_spec = pl.BlockSpec((NC, R, CW), lambda i: (0, i, 0))
_w_spec = pl.BlockSpec((D, D), lambda i: (0, 0))
_v_spec = pl.BlockSpec((1, D), lambda i: (0, 0))

_dense1 = pl.pallas_call(
    _dense1_body,
    grid=(N // R,),
    in_specs=[_row_spec, _part_spec, _cnt_spec, _w_spec, _v_spec, _w_spec,
              _v_spec, _v_spec, _v_spec, _v_spec],
    out_specs=_row_spec,
    out_shape=jax.ShapeDtypeStruct((N, D), jnp.float32),
)

_dense2 = pl.pallas_call(
    _dense2_body,
    grid=(N // R,),
    in_specs=[_row_spec, _part_spec, _cnt_spec, _w_spec, _v_spec, _w_spec],
    out_specs=_row_spec,
    out_shape=jax.ShapeDtypeStruct((N, D), jnp.float32),
)


def kernel(x, edge_index, W1l, b1, W1r, gamma, beta, run_mean, run_var,
           W2l, b2, W2r):
    src = edge_index[0].reshape(NW, NCHUNK, CHUNK)
    dst = edge_index[1].reshape(NW, NCHUNK, CHUNK)
    zfeat = jnp.zeros((RPS, D), jnp.float32)
    dstc = edge_index[1].reshape(NW, NCHUNKC, CHUNKC)
    zcnt = jnp.zeros((RPS, CW), jnp.float32)
    ones = jnp.ones((CHUNKC, CW), jnp.float32)

    c1 = _segcnt(dstc, zcnt, ones)
    s1 = _segsum(x, src, dst, zfeat)
    h = _dense1(x, s1, c1, W1l, b1.reshape(1, D), W1r,
                gamma.reshape(1, D), beta.reshape(1, D),
                run_mean.reshape(1, D), run_var.reshape(1, D))
    s2 = _segsum(h, src, dst, zfeat)
    out = _dense2(h, s2, c1, W2l, b2.reshape(1, D), W2r)
    return out
